# baseline (device time: 38386 ns/iter reference)
import jax
import jax.numpy as jnp
from jax import lax
from jax.experimental import pallas as pl
from jax.experimental.pallas import tpu as pltpu

B, SQ, H, D = 8, 1, 8, 64
SCALE = D ** -0.5


def kernel(Q, K, V):
    def body(q_ref, k_ref, v_ref, out_ref, send_buf, recv_buf, send_sem, recv_sem):
        my_x = lax.axis_index("x")
        my_y = lax.axis_index("y")
        my_z = lax.axis_index("z")

        q = q_ref[...]
        k = k_ref[...]
        v = v_ref[...]
        s = jnp.sum(q * k, axis=-1, keepdims=True) * SCALE
        m = jnp.max(s, axis=1, keepdims=True)
        p = jnp.exp(s - m)
        l = jnp.sum(p, axis=1, keepdims=True)
        o = jnp.sum(p * v, axis=1)

        send_buf[:, :, 0:D] = o
        send_buf[:, :, D:D + 1] = m.reshape(B, H, 1)
        send_buf[:, :, D + 1:D + 2] = l.reshape(B, H, 1)

        barrier_sem = pltpu.get_barrier_semaphore()
        pl.semaphore_signal(
            barrier_sem, inc=1,
            device_id=(my_x, my_y, 1 - my_z),
            device_id_type=pl.DeviceIdType.MESH,
        )
        pl.semaphore_wait(barrier_sem, 1)

        rdma = pltpu.make_async_remote_copy(
            src_ref=send_buf,
            dst_ref=recv_buf,
            send_sem=send_sem,
            recv_sem=recv_sem,
            device_id=(my_x, my_y, 1 - my_z),
            device_id_type=pl.DeviceIdType.MESH,
        )
        rdma.start()
        rdma.wait()

        o_b = recv_buf[:, :, 0:D]
        m_b = recv_buf[:, :, D:D + 1]
        l_b = recv_buf[:, :, D + 1:D + 2]
        m_a = m.reshape(B, H, 1)
        l_a = l.reshape(B, H, 1)
        m_new = jnp.maximum(m_a, m_b)
        alpha = jnp.exp(m_a - m_new)
        beta = jnp.exp(m_b - m_new)
        l_new = alpha * l_a + beta * l_b
        o_new = (alpha * o + beta * o_b) / l_new
        out_ref[...] = o_new.reshape(B, SQ, H, D)

    out_shape = jax.ShapeDtypeStruct((B, SQ, H, D), jnp.float32)
    return pl.pallas_call(
        body,
        out_shape=out_shape,
        in_specs=[
            pl.BlockSpec(memory_space=pltpu.VMEM),
            pl.BlockSpec(memory_space=pltpu.VMEM),
            pl.BlockSpec(memory_space=pltpu.VMEM),
        ],
        out_specs=pl.BlockSpec(memory_space=pltpu.VMEM),
        scratch_shapes=[
            pltpu.VMEM((B, H, 128), jnp.float32),
            pltpu.VMEM((B, H, 128), jnp.float32),
            pltpu.SemaphoreType.DMA,
            pltpu.SemaphoreType.DMA,
        ],
        compiler_params=pltpu.CompilerParams(collective_id=0),
    )(Q, K, V)
